# baseline (speedup 1.0000x reference)
import functools

import jax
import jax.numpy as jnp
from jax import lax
from jax.experimental import pallas as pl
from jax.experimental.pallas import tpu as pltpu
import jax.experimental.pallas.tpu_sc as plsc

_ROWS = 65536
_D = 6 * 7 * 8
_NC, _NS = 2, 16
_NW = _NC * _NS
_PER_W = _ROWS // _NW
_CHR = 256
_NCHUNKS = _PER_W // _CHR


def _t_kernel(t_ref):
    flat = (jax.lax.broadcasted_iota(jnp.int32, (2, _D), 0) * _D
            + jax.lax.broadcasted_iota(jnp.int32, (2, _D), 1))
    t_ref[...] = flat.astype(jnp.float32)


@functools.partial(
    pl.kernel,
    out_type=jax.ShapeDtypeStruct((_ROWS, _D), jnp.float32),
    mesh=plsc.VectorSubcoreMesh(core_axis_name="c", subcore_axis_name="s"),
    scratch_types=[
        pltpu.VMEM((_CHR, _D), jnp.float32),
    ],
)
def _sc_copy(x_hbm, t_hbm, o_hbm, buf):
    wid = lax.axis_index("s") * _NC + lax.axis_index("c")
    base = wid * _PER_W
    for i in range(_NCHUNKS):
        off = base + i * _CHR
        pltpu.sync_copy(x_hbm.at[pl.ds(off, _CHR)], buf)

        if i == 0:
            @pl.when(wid == 0)
            def _():
                pltpu.sync_copy(t_hbm, buf.at[pl.ds(0, 2)])

        pltpu.sync_copy(buf, o_hbm.at[pl.ds(off, _CHR)])


def kernel(x):
    t = pl.pallas_call(
        _t_kernel,
        out_shape=jax.ShapeDtypeStruct((2, _D), jnp.float32),
    )()
    out = _sc_copy(x.reshape(_ROWS, _D), t)
    return out.reshape(x.shape)


# speedup vs baseline: 1.0044x; 1.0044x over previous
import functools

import jax
import jax.numpy as jnp
from jax import lax
from jax.experimental import pallas as pl
from jax.experimental.pallas import tpu as pltpu
import jax.experimental.pallas.tpu_sc as plsc

_ROWS = 65536
_D = 6 * 7 * 8
_NC, _NS = 2, 16
_NW = _NC * _NS
_PER_W = _ROWS // _NW
_CHR = 128
_NCHUNKS = _PER_W // _CHR


def _t_kernel(t_ref):
    flat = (jax.lax.broadcasted_iota(jnp.int32, (2, _D), 0) * _D
            + jax.lax.broadcasted_iota(jnp.int32, (2, _D), 1))
    t_ref[...] = flat.astype(jnp.float32)


@functools.partial(
    pl.kernel,
    out_type=jax.ShapeDtypeStruct((_ROWS, _D), jnp.float32),
    mesh=plsc.VectorSubcoreMesh(core_axis_name="c", subcore_axis_name="s"),
    scratch_types=[
        pltpu.VMEM((_CHR, _D), jnp.float32),
        pltpu.VMEM((_CHR, _D), jnp.float32),
        pltpu.SemaphoreType.DMA((2,)),
        pltpu.SemaphoreType.DMA((2,)),
    ],
)
def _sc_copy(x_hbm, t_hbm, o_hbm, buf0, buf1, sem_in, sem_out):
    wid = lax.axis_index("s") * _NC + lax.axis_index("c")
    base = wid * _PER_W
    bufs = (buf0, buf1)

    def mk_in(i):
        return pltpu.async_copy(
            x_hbm.at[pl.ds(base + i * _CHR, _CHR)], bufs[i % 2],
            sem_in.at[i % 2])

    def mk_out(i):
        return pltpu.async_copy(
            bufs[i % 2], o_hbm.at[pl.ds(base + i * _CHR, _CHR)],
            sem_out.at[i % 2])

    ins = [None] * _NCHUNKS
    outs = [None] * _NCHUNKS
    ins[0] = mk_in(0)
    for i in range(_NCHUNKS):
        if i + 1 < _NCHUNKS:
            if i >= 1:
                outs[i - 1].wait()
            ins[i + 1] = mk_in(i + 1)
        ins[i].wait()

        if i == 0:
            @pl.when(wid == 0)
            def _():
                pltpu.sync_copy(t_hbm, bufs[0].at[pl.ds(0, 2)])

        outs[i] = mk_out(i)
    outs[_NCHUNKS - 1].wait()
    outs[_NCHUNKS - 2].wait()


def kernel(x):
    t = pl.pallas_call(
        _t_kernel,
        out_shape=jax.ShapeDtypeStruct((2, _D), jnp.float32),
    )()
    out = _sc_copy(x.reshape(_ROWS, _D), t)
    return out.reshape(x.shape)


# speedup vs baseline: 1.0301x; 1.0256x over previous
import functools

import jax
import jax.numpy as jnp
from jax import lax
from jax.experimental import pallas as pl
from jax.experimental.pallas import tpu as pltpu
import jax.experimental.pallas.tpu_sc as plsc

_ROWS = 65536
_D = 6 * 7 * 8
_HEAD = 8
_BLOCK = 8192
_LANE = 16


@functools.partial(
    pl.kernel,
    out_type=jax.ShapeDtypeStruct((_HEAD, _D), jnp.float32),
    mesh=plsc.VectorSubcoreMesh(core_axis_name="c", subcore_axis_name="s"),
    scratch_types=[
        pltpu.VMEM((_HEAD, _D), jnp.float32),
    ],
)
def _sc_head(x_hbm, head_hbm, buf):
    wid = lax.axis_index("s") * 2 + lax.axis_index("c")

    @pl.when(wid == 0)
    def _():
        pltpu.sync_copy(x_hbm.at[pl.ds(0, _HEAD)], buf)
        for k in range(2 * _D // _LANE):
            r, col = divmod(k * _LANE, _D)
            vec = (jnp.arange(_LANE, dtype=jnp.int32)
                   + k * _LANE).astype(jnp.float32)
            buf[r, pl.ds(col, _LANE)] = vec
        pltpu.sync_copy(buf, head_hbm)


def _tc_copy(x_ref, head_ref, o_ref):
    o_ref[...] = x_ref[...]

    @pl.when(pl.program_id(0) == 0)
    def _():
        o_ref[0:_HEAD, :] = head_ref[...]


def kernel(x):
    xf = x.reshape(_ROWS, _D)
    head = _sc_head(xf)
    out = pl.pallas_call(
        _tc_copy,
        grid=(_ROWS // _BLOCK,),
        in_specs=[
            pl.BlockSpec((_BLOCK, _D), lambda i: (i, 0)),
            pl.BlockSpec((_HEAD, _D), lambda i: (0, 0)),
        ],
        out_specs=pl.BlockSpec((_BLOCK, _D), lambda i: (i, 0)),
        out_shape=jax.ShapeDtypeStruct((_ROWS, _D), jnp.float32),
    )(xf, head)
    return out.reshape(x.shape)


# speedup vs baseline: 1.0375x; 1.0073x over previous
import functools

import jax
import jax.numpy as jnp
from jax import lax
from jax.experimental import pallas as pl
from jax.experimental.pallas import tpu as pltpu
import jax.experimental.pallas.tpu_sc as plsc

_ROWS = 65536
_D = 6 * 7 * 8
_HEAD = 8
_BLOCK = 8192
_LANE = 16


@functools.partial(
    pl.kernel,
    out_type=jax.ShapeDtypeStruct((_HEAD, _D), jnp.float32),
    mesh=plsc.VectorSubcoreMesh(core_axis_name="c", subcore_axis_name="s",
                                num_cores=1, num_subcores=1),
    scratch_types=[
        pltpu.VMEM((_HEAD, _D), jnp.float32),
    ],
)
def _sc_head(x_hbm, head_hbm, buf):
    wid = lax.axis_index("s") * 2 + lax.axis_index("c")

    @pl.when(wid == 0)
    def _():
        pltpu.sync_copy(x_hbm.at[pl.ds(0, _HEAD)], buf)
        for k in range(2 * _D // _LANE):
            r, col = divmod(k * _LANE, _D)
            vec = (jnp.arange(_LANE, dtype=jnp.int32)
                   + k * _LANE).astype(jnp.float32)
            buf[r, pl.ds(col, _LANE)] = vec
        pltpu.sync_copy(buf, head_hbm)


def _tc_copy(x_ref, head_ref, o_ref):
    o_ref[...] = x_ref[...]

    @pl.when(pl.program_id(0) == 0)
    def _():
        o_ref[0:_HEAD, :] = head_ref[...]


def kernel(x):
    xf = x.reshape(_ROWS, _D)
    head = _sc_head(xf)
    out = pl.pallas_call(
        _tc_copy,
        grid=(_ROWS // _BLOCK,),
        in_specs=[
            pl.BlockSpec((_BLOCK, _D), lambda i: (i, 0)),
            pl.BlockSpec((_HEAD, _D), lambda i: (0, 0)),
        ],
        out_specs=pl.BlockSpec((_BLOCK, _D), lambda i: (i, 0)),
        out_shape=jax.ShapeDtypeStruct((_ROWS, _D), jnp.float32),
    )(xf, head)
    return out.reshape(x.shape)


# speedup vs baseline: 1.0415x; 1.0038x over previous
import functools

import jax
import jax.numpy as jnp
from jax import lax
from jax.experimental import pallas as pl
from jax.experimental.pallas import tpu as pltpu
import jax.experimental.pallas.tpu_sc as plsc

_ROWS = 65536
_D = 6 * 7 * 8
_HEAD = 8
_BLOCK = 8192
_LANE = 16


@functools.partial(
    pl.kernel,
    out_type=jax.ShapeDtypeStruct((_HEAD, _D), jnp.float32),
    mesh=plsc.VectorSubcoreMesh(core_axis_name="c", subcore_axis_name="s",
                                num_cores=1, num_subcores=1),
    scratch_types=[
        pltpu.VMEM((_HEAD, _D), jnp.float32),
    ],
)
def _sc_head(x_hbm, head_hbm, buf):
    pltpu.sync_copy(x_hbm.at[pl.ds(0, _HEAD)], buf)
    for k in range(2 * _D // _LANE):
        r, col = divmod(k * _LANE, _D)
        vec = (jnp.arange(_LANE, dtype=jnp.int32)
               + k * _LANE).astype(jnp.float32)
        buf[r, pl.ds(col, _LANE)] = vec
    pltpu.sync_copy(buf, head_hbm)


def _tc_copy(x_ref, o_ref):
    o_ref[...] = x_ref[...]


def _tc_patch(o_in_ref, head_ref, o_ref):
    o_ref[...] = head_ref[...]


def kernel(x):
    xf = x.reshape(_ROWS, _D)
    head = _sc_head(xf)
    bulk = pl.pallas_call(
        _tc_copy,
        grid=(_ROWS // _BLOCK,),
        in_specs=[pl.BlockSpec((_BLOCK, _D), lambda i: (i, 0))],
        out_specs=pl.BlockSpec((_BLOCK, _D), lambda i: (i, 0)),
        out_shape=jax.ShapeDtypeStruct((_ROWS, _D), jnp.float32),
    )(xf)
    out = pl.pallas_call(
        _tc_patch,
        grid=(1,),
        in_specs=[
            pl.BlockSpec((_HEAD, _D), lambda i: (0, 0)),
            pl.BlockSpec((_HEAD, _D), lambda i: (0, 0)),
        ],
        out_specs=pl.BlockSpec((_HEAD, _D), lambda i: (0, 0)),
        out_shape=jax.ShapeDtypeStruct((_ROWS, _D), jnp.float32),
        input_output_aliases={0: 0},
    )(bulk, head)
    return out.reshape(x.shape)
